# baseline (speedup 1.0000x reference)
import functools

import jax
import jax.numpy as jnp
from jax import lax
from jax.experimental import pallas as pl
from jax.experimental.pallas import tpu as pltpu
from jax.experimental.pallas import tpu_sc as plsc

HIDDEN = 64
BATCH = 16384
NC, NS = 2, 16
NW = NC * NS
BPW = BATCH // NW
CHUNK = 128
NCH = BPW // CHUNK


def _gather_sc(idx, p_table, c_table):
    mesh = plsc.VectorSubcoreMesh(core_axis_name="c", subcore_axis_name="s")

    @functools.partial(
        pl.kernel,
        mesh=mesh,
        compiler_params=pltpu.CompilerParams(use_tc_tiling_on_sc=False),
        out_type=(
            jax.ShapeDtypeStruct((BATCH, HIDDEN), jnp.float32),
            jax.ShapeDtypeStruct((BATCH, HIDDEN), jnp.float32),
        ),
        scratch_types=[
            pltpu.VMEM((NCH, CHUNK), jnp.int32),
            pltpu.VMEM((BPW, HIDDEN), jnp.float32),
            pltpu.VMEM((BPW, HIDDEN), jnp.float32),
            pltpu.SemaphoreType.DMA,
            pltpu.SemaphoreType.DMA,
        ],
    )
    def k(idx_hbm, p_hbm, c_hbm, p_out, c_out, idx_v, p_rows, c_rows, sem_p, sem_c):
        wid = lax.axis_index("c") * NS + lax.axis_index("s")
        base = wid * BPW
        for j in range(NCH):
            pltpu.sync_copy(idx_hbm.at[pl.ds(base + j * CHUNK, CHUNK)], idx_v.at[j])
        waits = []
        for j in range(NCH):
            dst = pl.ds(j * CHUNK, CHUNK)
            waits.append(
                pltpu.async_copy(p_hbm.at[idx_v.at[j]], p_rows.at[dst], sem_p))
            waits.append(
                pltpu.async_copy(c_hbm.at[idx_v.at[j]], c_rows.at[dst], sem_c))
        for w in waits:
            w.wait()
        pltpu.sync_copy(p_rows, p_out.at[pl.ds(base, BPW)])
        pltpu.sync_copy(c_rows, c_out.at[pl.ds(base, BPW)])

    return k(idx, p_table, c_table)


def _rating_tc(p, c, W, b):
    blk = 2048

    def body(p_ref, c_ref, w_ref, b_ref, o_ref):
        s = jnp.sum((p_ref[...] + c_ref[...]) * w_ref[...], axis=1, keepdims=True)
        o_ref[...] = jax.nn.sigmoid(s + b_ref[...])

    return pl.pallas_call(
        body,
        grid=(BATCH // blk,),
        in_specs=[
            pl.BlockSpec((blk, HIDDEN), lambda i: (i, 0)),
            pl.BlockSpec((blk, HIDDEN), lambda i: (i, 0)),
            pl.BlockSpec((1, HIDDEN), lambda i: (0, 0)),
            pl.BlockSpec((1, 1), lambda i: (0, 0)),
        ],
        out_specs=pl.BlockSpec((blk, 1), lambda i: (i, 0)),
        out_shape=jax.ShapeDtypeStruct((BATCH, 1), jnp.float32),
    )(p, c, W, b.reshape(1, 1))


def kernel(item_indices, item_personality_table, item_commonality_table, W, b):
    idx = item_indices.astype(jnp.int32)
    p, c = _gather_sc(idx, item_personality_table, item_commonality_table)
    rating = _rating_tc(p, c, W, b)
    return (rating, p, c)


# speedup vs baseline: 1.0707x; 1.0707x over previous
import functools

import jax
import jax.numpy as jnp
from jax import lax
from jax.experimental import pallas as pl
from jax.experimental.pallas import tpu as pltpu
from jax.experimental.pallas import tpu_sc as plsc

HIDDEN = 64
ROW = 128
BATCH = 16384
NC, NS = 2, 16
NW = NC * NS
BPW = BATCH // NW
CHUNK = 128
NCH = BPW // CHUNK


def _gather_sc(idx, p2, c2):
    mesh = plsc.VectorSubcoreMesh(core_axis_name="c", subcore_axis_name="s")

    @functools.partial(
        pl.kernel,
        mesh=mesh,
        out_type=(
            jax.ShapeDtypeStruct((BATCH, ROW), jnp.float32),
            jax.ShapeDtypeStruct((BATCH, ROW), jnp.float32),
        ),
        scratch_types=[
            pltpu.VMEM((NCH, CHUNK), jnp.int32),
            pltpu.VMEM((BPW, ROW), jnp.float32),
            pltpu.SemaphoreType.DMA,
        ],
    )
    def k(idx_hbm, p_hbm, c_hbm, p_out, c_out, idx_v, rows_v, sem):
        wid = lax.axis_index("c") * NS + lax.axis_index("s")
        base = wid * BPW
        for j in range(NCH):
            pltpu.sync_copy(idx_hbm.at[pl.ds(base + j * CHUNK, CHUNK)],
                            idx_v.at[j])
        for (tab, out) in ((p_hbm, p_out), (c_hbm, c_out)):
            waits = []
            for j in range(NCH):
                waits.append(pltpu.async_copy(
                    tab.at[idx_v.at[j]],
                    rows_v.at[pl.ds(j * CHUNK, CHUNK)], sem))
            for w in waits:
                w.wait()
            pltpu.sync_copy(rows_v, out.at[pl.ds(base, BPW)])

    return k(idx, p2, c2)


def _rating_tc(p128, c128, W, b):
    blk = 2048

    def body(p_ref, c_ref, w_ref, b_ref, o_ref):
        pc = p_ref[...] + c_ref[...]
        s = jnp.sum(pc[:, :HIDDEN] * w_ref[...], axis=1, keepdims=True)
        o_ref[...] = jax.nn.sigmoid(s + b_ref[...])

    return pl.pallas_call(
        body,
        grid=(BATCH // blk,),
        in_specs=[
            pl.BlockSpec((blk, ROW), lambda i: (i, 0)),
            pl.BlockSpec((blk, ROW), lambda i: (i, 0)),
            pl.BlockSpec((1, HIDDEN), lambda i: (0, 0)),
            pl.BlockSpec((1, 1), lambda i: (0, 0)),
        ],
        out_specs=pl.BlockSpec((blk, 1), lambda i: (i, 0)),
        out_shape=jax.ShapeDtypeStruct((BATCH, 1), jnp.float32),
    )(p128, c128, W, b.reshape(1, 1))


def kernel(item_indices, item_personality_table, item_commonality_table, W, b):
    idx = item_indices.astype(jnp.int32)
    p2 = jnp.pad(item_personality_table, ((0, 0), (0, ROW - HIDDEN)))
    c2 = jnp.pad(item_commonality_table, ((0, 0), (0, ROW - HIDDEN)))
    p128, c128 = _gather_sc(idx, p2, c2)
    rating = _rating_tc(p128, c128, W, b)
    return (rating, p128[:, :HIDDEN], c128[:, :HIDDEN])
